# baseline (speedup 1.0000x reference)
import functools

import numpy as np
import jax
import jax.numpy as jnp
from jax import lax
from jax.experimental import pallas as pl
from jax.experimental.pallas import tpu as pltpu
from jax.experimental.pallas import tpu_sc as plsc

N = 512
EMB = 64
RNN = 128
GRID = 8
GG = GRID * GRID
NMIX = 20
OUTD = NMIX * 6
NEIGH = 0.4

NC, NS, L = 2, 16, 16
NW = NC * NS
TPW = N // NW
PROWS = GG * N
PPAD = PROWS + N
CH = 64
PW = 128


def _prep_body(h_ref, wr_ref, xoff_ref, wemb_ref, bemb_ref,
               xsc_ref, ysc_ref, xsr_ref, ysr_ref,
               p_ref, emb_ref, ridx_ref):
    c = pl.program_id(0)

    @pl.when(c < GG)
    def _():
        p_ref[...] = jnp.dot(h_ref[...], wr_ref[0],
                             preferred_element_type=jnp.float32)

    @pl.when(c == GG)
    def _():
        p_ref[...] = jnp.zeros_like(p_ref)

    @pl.when(c == 0)
    def _():
        xo = xoff_ref[...]
        w = wemb_ref[...]
        emb = xo[:, 0:1] * w[0:1, :] + xo[:, 1:2] * w[1:2, :] + bemb_ref[...]
        emb_ref[...] = jnp.maximum(emb, 0.0)

        dx = xsr_ref[...] - (xsc_ref[...] - NEIGH / 2.0)
        dy = ysr_ref[...] - (ysc_ref[...] - NEIGH / 2.0)
        within = (dx >= 0.0) & (dx < NEIGH) & (dy >= 0.0) & (dy < NEIGH)
        cellx = jnp.floor(dx / NEIGH * GRID).astype(jnp.int32)
        celly = jnp.floor(dy / NEIGH * GRID).astype(jnp.int32)
        valid_cell = ((cellx >= 0) & (cellx < GRID)
                      & (celly >= 0) & (celly < GRID))
        idxc = jnp.clip(cellx + celly * GRID, 0, GG - 1)
        col = lax.broadcasted_iota(jnp.int32, (N, N), 1)
        row = lax.broadcasted_iota(jnp.int32, (N, N), 0)
        valid = within & valid_cell & (col != row)
        ridx_ref[...] = jnp.where(valid, idxc * N + col, PROWS)


def _prep(h, wr, xoff, wemb, bemb, xsc, ysc, xsr, ysr):
    full = lambda s: pl.BlockSpec(s, lambda c: (0,) * len(s))
    return pl.pallas_call(
        _prep_body,
        grid=(GG + 1,),
        in_specs=[
            full((N, RNN)),
            pl.BlockSpec((1, RNN, PW), lambda c: (jnp.minimum(c, GG - 1), 0, 0)),
            full((N, 2)),
            full((2, EMB)),
            full((1, EMB)),
            full((N, 1)), full((N, 1)), full((1, N)), full((1, N)),
        ],
        out_specs=[
            pl.BlockSpec((N, PW), lambda c: (c, 0)),
            pl.BlockSpec((N, EMB), lambda c: (0, 0)),
            pl.BlockSpec((N, N), lambda c: (0, 0)),
        ],
        out_shape=[
            jax.ShapeDtypeStruct((PPAD, PW), jnp.float32),
            jax.ShapeDtypeStruct((N, EMB), jnp.float32),
            jax.ShapeDtypeStruct((N, N), jnp.int32),
        ],
    )(h, wr, xoff, wemb, bemb, xsc, ysc, xsr, ysr)


def _pool_body(ridx_hbm, p_hbm, out_hbm, rid_v, idx_v, rows_v, acc_v, sem):
    wid = lax.axis_index("s") * NC + lax.axis_index("c")
    base = wid * TPW
    pltpu.sync_copy(ridx_hbm.at[pl.ds(base, TPW)], rid_v)

    for t in range(TPW):
        def cbody(ch, cnt):
            r = rid_v[t, pl.ds(ch * L, L)]
            m = r < PROWS
            plsc.store_compressed(idx_v.at[pl.ds(cnt, L)], r, mask=m)
            return cnt + jnp.sum(m.astype(jnp.int32))

        cnt = lax.fori_loop(0, N // L, cbody, 0, unroll=False)

        pad = jnp.full((L,), PROWS, jnp.int32)
        for q in range(CH // L):
            idx_v[pl.ds(cnt + q * L, L)] = pad

        nit = (cnt + CH - 1) // CH

        def gbody(g, accs):
            a0, a1, a2, a3 = accs
            desc = pltpu.async_copy(
                p_hbm.at[idx_v.at[pl.ds(g * CH, CH)]], rows_v, sem)
            desc.wait()

            def abody(r, accs2):
                b0, b1, b2, b3 = accs2
                return (b0 + rows_v[r, pl.ds(0, L)],
                        b1 + rows_v[r, pl.ds(L, L)],
                        b2 + rows_v[r, pl.ds(2 * L, L)],
                        b3 + rows_v[r, pl.ds(3 * L, L)])

            return lax.fori_loop(0, CH, abody, (a0, a1, a2, a3),
                                 unroll=False)

        zero = jnp.zeros((L,), jnp.float32)
        a0, a1, a2, a3 = lax.fori_loop(0, nit, gbody,
                                       (zero, zero, zero, zero),
                                       unroll=False)
        acc_v[t, pl.ds(0, L)] = a0
        acc_v[t, pl.ds(L, L)] = a1
        acc_v[t, pl.ds(2 * L, L)] = a2
        acc_v[t, pl.ds(3 * L, L)] = a3

    pltpu.sync_copy(acc_v, out_hbm.at[pl.ds(base, TPW)])


def _pool(ridx, p):
    mesh = plsc.VectorSubcoreMesh(core_axis_name="c", subcore_axis_name="s",
                                  num_cores=NC, num_subcores=NS)
    return pl.kernel(
        _pool_body,
        out_type=jax.ShapeDtypeStruct((N, EMB), jnp.float32),
        mesh=mesh,
        compiler_params=pltpu.CompilerParams(needs_layout_passes=False),
        scratch_types=[
            pltpu.VMEM((TPW, N), jnp.int32),
            pltpu.VMEM((N + 2 * CH,), jnp.int32),
            pltpu.VMEM((CH, PW), jnp.float32),
            pltpu.VMEM((TPW, EMB), jnp.float32),
            pltpu.SemaphoreType.DMA,
        ],
    )(ridx, p)


def _lstm_body(emb_ref, pool_ref, h_ref, c_ref, wih_ref, whh_ref,
               bias_ref, bsoc_ref, wout_ref, bout_ref, out_ref):
    hp = jnp.maximum(pool_ref[...] + bsoc_ref[...], 0.0)
    lstm_in = jnp.concatenate([emb_ref[...], hp], axis=1)
    gates = (jnp.dot(lstm_in, wih_ref[...], preferred_element_type=jnp.float32)
             + jnp.dot(h_ref[...], whh_ref[...],
                       preferred_element_type=jnp.float32)
             + bias_ref[...])
    i_g = gates[:, 0:RNN]
    f_g = gates[:, RNN:2 * RNN]
    g_g = gates[:, 2 * RNN:3 * RNN]
    o_g = gates[:, 3 * RNN:4 * RNN]
    c_new = (jax.nn.sigmoid(f_g) * c_ref[...]
             + jax.nn.sigmoid(i_g) * jnp.tanh(g_g))
    h_new = jax.nn.sigmoid(o_g) * jnp.tanh(c_new)
    out_ref[...] = (jnp.dot(h_new, wout_ref[...],
                            preferred_element_type=jnp.float32)
                    + bout_ref[...])


def _lstm(emb, pool, h, c, wih_t, whh_t, bias, bsoc, wout_p, bout_p):
    return pl.pallas_call(
        _lstm_body,
        out_shape=jax.ShapeDtypeStruct((N, 128), jnp.float32),
    )(emb, pool, h, c, wih_t, whh_t, bias, bsoc, wout_p, bout_p)


def kernel(xoff, xabs, h0, c0, W_embed, b_embed, W_social, b_social,
           W_ih, W_hh, b_ih, b_hh, W_out, b_out):
    h = h0[0]
    c = c0[0]
    wr = jnp.pad(W_social.reshape(GG, RNN, EMB), ((0, 0), (0, 0), (0, PW - EMB)))
    xsc = xabs[:, 0:1]
    ysc = xabs[:, 1:2]
    xsr = xabs[:, 0].reshape(1, N)
    ysr = xabs[:, 1].reshape(1, N)

    p, emb, ridx = _prep(h, wr, xoff, W_embed, b_embed.reshape(1, EMB),
                         xsc, ysc, xsr, ysr)

    pool = _pool(ridx, p)

    bias = (b_ih + b_hh).reshape(1, 4 * RNN)
    wout_p = jnp.pad(W_out, ((0, 0), (0, 128 - OUTD)))
    bout_p = jnp.pad(b_out, (0, 128 - OUTD)).reshape(1, 128)
    final = _lstm(emb, pool, h, c, W_ih.T, W_hh.T, bias,
                  b_social.reshape(1, EMB), wout_p, bout_p)[:, :OUTD]

    mu1, mu2, log_s1, log_s2, rho, pi = jnp.split(final, 6, axis=1)
    return (mu1, mu2, log_s1, log_s2, rho, pi)


# speedup vs baseline: 1.8199x; 1.8199x over previous
import functools

import numpy as np
import jax
import jax.numpy as jnp
from jax import lax
from jax.experimental import pallas as pl
from jax.experimental.pallas import tpu as pltpu
from jax.experimental.pallas import tpu_sc as plsc

N = 512
EMB = 64
RNN = 128
GRID = 8
GG = GRID * GRID
NMIX = 20
OUTD = NMIX * 6
NEIGH = 0.4

NC, NS, L = 2, 16, 16
NW = NC * NS
TPW = N // NW
PROWS = GG * N
PPAD = PROWS + N
CH = 32
NB = 4
MW = N + 2 * L + 96
TRASH = N + CH + L
IW = 640
PW = 128


def _prep_body(h_ref, wr_ref, xoff_ref, wemb_ref, bemb_ref,
               xsc_ref, ysc_ref, xsr_ref, ysr_ref,
               p_ref, emb_ref, ridx_ref, meta_ref):
    c = pl.program_id(0)

    @pl.when(c < GG)
    def _():
        p_ref[...] = jnp.dot(h_ref[...], wr_ref[0],
                             preferred_element_type=jnp.float32)

    @pl.when(c == GG)
    def _():
        p_ref[...] = jnp.zeros_like(p_ref)

    @pl.when(c == 0)
    def _():
        xo = xoff_ref[...]
        w = wemb_ref[...]
        emb = xo[:, 0:1] * w[0:1, :] + xo[:, 1:2] * w[1:2, :] + bemb_ref[...]
        emb_ref[...] = jnp.maximum(emb, 0.0)

        dx = xsr_ref[...] - (xsc_ref[...] - NEIGH / 2.0)
        dy = ysr_ref[...] - (ysc_ref[...] - NEIGH / 2.0)
        within = (dx >= 0.0) & (dx < NEIGH) & (dy >= 0.0) & (dy < NEIGH)
        cellx = jnp.floor(dx / NEIGH * GRID).astype(jnp.int32)
        celly = jnp.floor(dy / NEIGH * GRID).astype(jnp.int32)
        valid_cell = ((cellx >= 0) & (cellx < GRID)
                      & (celly >= 0) & (celly < GRID))
        idxc = jnp.clip(cellx + celly * GRID, 0, GG - 1)
        col = lax.broadcasted_iota(jnp.int32, (N, N), 1)
        row = lax.broadcasted_iota(jnp.int32, (N, N), 0)
        valid = within & valid_cell & (col != row)
        ridx_ref[...] = jnp.where(valid, idxc * N + col, PROWS)

        vf = valid.astype(jnp.float32)
        tri = (lax.broadcasted_iota(jnp.int32, (N, MW), 0)
               < lax.broadcasted_iota(jnp.int32, (N, MW), 1)
               ).astype(jnp.float32)
        pos = jnp.dot(vf, tri, preferred_element_type=jnp.float32)
        posi = pos.astype(jnp.int32)
        meta_ref[...] = jnp.concatenate(
            [jnp.where(valid, posi[:, :N], TRASH), posi[:, N:]], axis=1)


def _prep(h, wr, xoff, wemb, bemb, xsc, ysc, xsr, ysr):
    full = lambda s: pl.BlockSpec(s, lambda c: (0,) * len(s))
    return pl.pallas_call(
        _prep_body,
        grid=(GG + 1,),
        in_specs=[
            full((N, RNN)),
            pl.BlockSpec((1, RNN, PW), lambda c: (jnp.minimum(c, GG - 1), 0, 0)),
            full((N, 2)),
            full((2, EMB)),
            full((1, EMB)),
            full((N, 1)), full((N, 1)), full((1, N)), full((1, N)),
        ],
        out_specs=[
            pl.BlockSpec((N, PW), lambda c: (c, 0)),
            pl.BlockSpec((N, EMB), lambda c: (0, 0)),
            pl.BlockSpec((N, N), lambda c: (0, 0)),
            pl.BlockSpec((N, MW), lambda c: (0, 0)),
        ],
        out_shape=[
            jax.ShapeDtypeStruct((PPAD, PW), jnp.float32),
            jax.ShapeDtypeStruct((N, EMB), jnp.float32),
            jax.ShapeDtypeStruct((N, N), jnp.int32),
            jax.ShapeDtypeStruct((N, MW), jnp.int32),
        ],
    )(h, wr, xoff, wemb, bemb, xsc, ysc, xsr, ysr)


def _pool_body(ridx_hbm, meta_hbm, p_hbm, out_hbm,
               rid_v, pos_v, idxa_v, idxb_v, rows_v, acc_v, sem):
    wid = lax.axis_index("s") * NC + lax.axis_index("c")
    base = wid * TPW
    pltpu.sync_copy(ridx_hbm.at[pl.ds(base, TPW)], rid_v)
    pltpu.sync_copy(meta_hbm.at[pl.ds(base, TPW)], pos_v)

    idxbufs = [idxa_v, idxb_v]
    pad = jnp.full((L,), PROWS, jnp.int32)

    def compact(t, ib):
        def cbody(ch, _):
            r = rid_v[t, pl.ds(ch * L, L)]
            pv = pos_v[t, pl.ds(ch * L, L)]
            plsc.store_scatter(ib, [pv], r)
            return 0

        lax.fori_loop(0, N // L, cbody, 0, unroll=False)
        cnt = pos_v[t, pl.ds(N, L)][0]
        for q in range(CH // L):
            ib[pl.ds(cnt + q * L, L)] = pad
        return jnp.right_shift(cnt + CH - 1, 5)

    def fire(ib, g, b):
        pltpu.async_copy(p_hbm.at[ib.at[pl.ds(g * CH, CH)]],
                         rows_v.at[b], sem)

    def drain(t, nit, ib):
        def gbody(g, accs):
            pltpu.make_async_copy(p_hbm.at[ib.at[pl.ds(0, CH)]],
                                  rows_v.at[0], sem).wait()
            b = jnp.bitwise_and(g, NB - 1)

            @pl.when(g + NB < nit)
            def _():
                fire(ib, g + NB, b)

            def abody(r, accs2):
                b0, b1, b2, b3 = accs2
                return (b0 + rows_v[b, r, pl.ds(0, L)],
                        b1 + rows_v[b, r, pl.ds(L, L)],
                        b2 + rows_v[b, r, pl.ds(2 * L, L)],
                        b3 + rows_v[b, r, pl.ds(3 * L, L)])

            return lax.fori_loop(0, CH, abody, accs, unroll=False)

        zero = jnp.zeros((L,), jnp.float32)
        a0, a1, a2, a3 = lax.fori_loop(0, nit, gbody,
                                       (zero, zero, zero, zero),
                                       unroll=False)
        acc_v[t, pl.ds(0, L)] = a0
        acc_v[t, pl.ds(L, L)] = a1
        acc_v[t, pl.ds(2 * L, L)] = a2
        acc_v[t, pl.ds(3 * L, L)] = a3

    prev = None
    for t in range(TPW):
        ib = idxbufs[t & 1]
        nit = compact(t, ib)
        if prev is not None:
            drain(*prev)
        for b in range(NB):
            @pl.when(b < nit)
            def _(b=b, ib=ib):
                fire(ib, b, b)
        prev = (t, nit, ib)
    drain(*prev)

    pltpu.sync_copy(acc_v, out_hbm.at[pl.ds(base, TPW)])


def _pool(ridx, meta, p):
    mesh = plsc.VectorSubcoreMesh(core_axis_name="c", subcore_axis_name="s",
                                  num_cores=NC, num_subcores=NS)
    return pl.kernel(
        _pool_body,
        out_type=jax.ShapeDtypeStruct((N, EMB), jnp.float32),
        mesh=mesh,
        compiler_params=pltpu.CompilerParams(needs_layout_passes=False),
        scratch_types=[
            pltpu.VMEM((TPW, N), jnp.int32),
            pltpu.VMEM((TPW, MW), jnp.int32),
            pltpu.VMEM((IW,), jnp.int32),
            pltpu.VMEM((IW,), jnp.int32),
            pltpu.VMEM((NB, CH, PW), jnp.float32),
            pltpu.VMEM((TPW, EMB), jnp.float32),
            pltpu.SemaphoreType.DMA,
        ],
    )(ridx, meta, p)


def _lstm_body(emb_ref, pool_ref, h_ref, c_ref, wih_ref, whh_ref,
               bias_ref, bsoc_ref, wout_ref, bout_ref, out_ref):
    hp = jnp.maximum(pool_ref[...] + bsoc_ref[...], 0.0)
    lstm_in = jnp.concatenate([emb_ref[...], hp], axis=1)
    gates = (jnp.dot(lstm_in, wih_ref[...], preferred_element_type=jnp.float32)
             + jnp.dot(h_ref[...], whh_ref[...],
                       preferred_element_type=jnp.float32)
             + bias_ref[...])
    i_g = gates[:, 0:RNN]
    f_g = gates[:, RNN:2 * RNN]
    g_g = gates[:, 2 * RNN:3 * RNN]
    o_g = gates[:, 3 * RNN:4 * RNN]
    c_new = (jax.nn.sigmoid(f_g) * c_ref[...]
             + jax.nn.sigmoid(i_g) * jnp.tanh(g_g))
    h_new = jax.nn.sigmoid(o_g) * jnp.tanh(c_new)
    out_ref[...] = (jnp.dot(h_new, wout_ref[...],
                            preferred_element_type=jnp.float32)
                    + bout_ref[...])


def _lstm(emb, pool, h, c, wih_t, whh_t, bias, bsoc, wout_p, bout_p):
    return pl.pallas_call(
        _lstm_body,
        out_shape=jax.ShapeDtypeStruct((N, 128), jnp.float32),
    )(emb, pool, h, c, wih_t, whh_t, bias, bsoc, wout_p, bout_p)


def kernel(xoff, xabs, h0, c0, W_embed, b_embed, W_social, b_social,
           W_ih, W_hh, b_ih, b_hh, W_out, b_out):
    h = h0[0]
    c = c0[0]
    wr = jnp.pad(W_social.reshape(GG, RNN, EMB), ((0, 0), (0, 0), (0, PW - EMB)))
    xsc = xabs[:, 0:1]
    ysc = xabs[:, 1:2]
    xsr = xabs[:, 0].reshape(1, N)
    ysr = xabs[:, 1].reshape(1, N)

    p, emb, ridx, meta = _prep(h, wr, xoff, W_embed, b_embed.reshape(1, EMB),
                               xsc, ysc, xsr, ysr)

    pool = _pool(ridx, meta, p)

    bias = (b_ih + b_hh).reshape(1, 4 * RNN)
    wout_p = jnp.pad(W_out, ((0, 0), (0, 128 - OUTD)))
    bout_p = jnp.pad(b_out, (0, 128 - OUTD)).reshape(1, 128)
    final = _lstm(emb, pool, h, c, W_ih.T, W_hh.T, bias,
                  b_social.reshape(1, EMB), wout_p, bout_p)[:, :OUTD]

    mu1, mu2, log_s1, log_s2, rho, pi = jnp.split(final, 6, axis=1)
    return (mu1, mu2, log_s1, log_s2, rho, pi)


# speedup vs baseline: 6.4606x; 3.5500x over previous
import functools

import numpy as np
import jax
import jax.numpy as jnp
from jax import lax
from jax.experimental import pallas as pl
from jax.experimental.pallas import tpu as pltpu
from jax.experimental.pallas import tpu_sc as plsc

N = 512
EMB = 64
RNN = 128
GRID = 8
GG = GRID * GRID
NMIX = 20
OUTD = NMIX * 6
NEIGH = 0.4

NC, NS, L = 2, 16, 16
NW = NC * NS
TPW = N // NW
PROWS = GG * N
PPAD = PROWS + N
CH = 32
NB = 4
MW = N + 2 * L + 96
TRASH = N + CH + L
IW = 640
PW = 128


def _prep_body(h_ref, wr_ref, xoff_ref, wemb_ref, bemb_ref,
               xsc_ref, ysc_ref, xsr_ref, ysr_ref,
               p_ref, emb_ref, ridx_ref, meta_ref):
    c = pl.program_id(0)

    @pl.when(c < GG)
    def _():
        p_ref[...] = jnp.dot(h_ref[...], wr_ref[0],
                             preferred_element_type=jnp.float32)

    @pl.when(c == GG)
    def _():
        p_ref[...] = jnp.zeros_like(p_ref)

    @pl.when(c == 0)
    def _():
        xo = xoff_ref[...]
        w = wemb_ref[...]
        emb = xo[:, 0:1] * w[0:1, :] + xo[:, 1:2] * w[1:2, :] + bemb_ref[...]
        emb_ref[...] = jnp.maximum(emb, 0.0)

        dx = xsr_ref[...] - (xsc_ref[...] - NEIGH / 2.0)
        dy = ysr_ref[...] - (ysc_ref[...] - NEIGH / 2.0)
        within = (dx >= 0.0) & (dx < NEIGH) & (dy >= 0.0) & (dy < NEIGH)
        cellx = jnp.floor(dx / NEIGH * GRID).astype(jnp.int32)
        celly = jnp.floor(dy / NEIGH * GRID).astype(jnp.int32)
        valid_cell = ((cellx >= 0) & (cellx < GRID)
                      & (celly >= 0) & (celly < GRID))
        idxc = jnp.clip(cellx + celly * GRID, 0, GG - 1)
        col = lax.broadcasted_iota(jnp.int32, (N, N), 1)
        row = lax.broadcasted_iota(jnp.int32, (N, N), 0)
        valid = within & valid_cell & (col != row)
        ridx_ref[...] = jnp.where(valid, idxc * N + col, PROWS)

        vf = valid.astype(jnp.float32)
        tri = (lax.broadcasted_iota(jnp.int32, (N, MW), 0)
               < lax.broadcasted_iota(jnp.int32, (N, MW), 1)
               ).astype(jnp.float32)
        pos = jnp.dot(vf, tri, preferred_element_type=jnp.float32)
        posi = pos.astype(jnp.int32)
        meta_ref[...] = jnp.concatenate(
            [jnp.where(valid, posi[:, :N], TRASH), posi[:, N:]], axis=1)


def _prep(h, wr, xoff, wemb, bemb, xsc, ysc, xsr, ysr):
    full = lambda s: pl.BlockSpec(s, lambda c: (0,) * len(s))
    return pl.pallas_call(
        _prep_body,
        grid=(GG + 1,),
        in_specs=[
            full((N, RNN)),
            pl.BlockSpec((1, RNN, PW), lambda c: (jnp.minimum(c, GG - 1), 0, 0)),
            full((N, 2)),
            full((2, EMB)),
            full((1, EMB)),
            full((N, 1)), full((N, 1)), full((1, N)), full((1, N)),
        ],
        out_specs=[
            pl.BlockSpec((N, PW), lambda c: (c, 0)),
            pl.BlockSpec((N, EMB), lambda c: (0, 0)),
            pl.BlockSpec((N, N), lambda c: (0, 0)),
            pl.BlockSpec((N, MW), lambda c: (0, 0)),
        ],
        out_shape=[
            jax.ShapeDtypeStruct((PPAD, PW), jnp.float32),
            jax.ShapeDtypeStruct((N, EMB), jnp.float32),
            jax.ShapeDtypeStruct((N, N), jnp.int32),
            jax.ShapeDtypeStruct((N, MW), jnp.int32),
        ],
    )(h, wr, xoff, wemb, bemb, xsc, ysc, xsr, ysr)


def _pool_body(ridx_hbm, meta_hbm, p_hbm, out_hbm,
               rid_v, pos_v, idxa_v, idxb_v, rows_v, acc_v, sem):
    wid = lax.axis_index("s") * NC + lax.axis_index("c")
    base = wid * TPW
    pltpu.sync_copy(ridx_hbm.at[pl.ds(base, TPW)], rid_v)
    pltpu.sync_copy(meta_hbm.at[pl.ds(base, TPW)], pos_v)

    idxbufs = [idxa_v, idxb_v]
    def compact(t, ib):
        pad = jnp.full((L,), PROWS, jnp.int32) + (base + t)
        def cbody(ch, _):
            r = rid_v[t, pl.ds(ch * L, L)]
            pv = pos_v[t, pl.ds(ch * L, L)]
            plsc.store_scatter(ib, [pv], r)
            return 0

        lax.fori_loop(0, N // L, cbody, 0, unroll=False)
        cnt = pos_v[t, pl.ds(N, L)][0]
        for q in range(CH // L):
            ib[pl.ds(cnt + q * L, L)] = pad
        return jnp.right_shift(cnt + CH - 1, 5)

    def fire(ib, g, b):
        pltpu.async_copy(p_hbm.at[ib.at[pl.ds(g * CH, CH)]],
                         rows_v.at[b], sem)

    def drain(t, nit, ib):
        def gbody(g, accs):
            pltpu.make_async_copy(p_hbm.at[ib.at[pl.ds(0, CH)]],
                                  rows_v.at[0], sem).wait()
            b = jnp.bitwise_and(g, NB - 1)

            @pl.when(g + NB < nit)
            def _():
                fire(ib, g + NB, b)

            def abody(r, accs2):
                b0, b1, b2, b3 = accs2
                return (b0 + rows_v[b, r, pl.ds(0, L)],
                        b1 + rows_v[b, r, pl.ds(L, L)],
                        b2 + rows_v[b, r, pl.ds(2 * L, L)],
                        b3 + rows_v[b, r, pl.ds(3 * L, L)])

            return lax.fori_loop(0, CH, abody, accs, unroll=False)

        zero = jnp.zeros((L,), jnp.float32)
        a0, a1, a2, a3 = lax.fori_loop(0, nit, gbody,
                                       (zero, zero, zero, zero),
                                       unroll=False)
        acc_v[t, pl.ds(0, L)] = a0
        acc_v[t, pl.ds(L, L)] = a1
        acc_v[t, pl.ds(2 * L, L)] = a2
        acc_v[t, pl.ds(3 * L, L)] = a3

    prev = None
    for t in range(TPW):
        ib = idxbufs[t & 1]
        nit = compact(t, ib)
        if prev is not None:
            drain(*prev)
        for b in range(NB):
            @pl.when(b < nit)
            def _(b=b, ib=ib):
                fire(ib, b, b)
        prev = (t, nit, ib)
    drain(*prev)

    pltpu.sync_copy(acc_v, out_hbm.at[pl.ds(base, TPW)])


def _pool(ridx, meta, p):
    mesh = plsc.VectorSubcoreMesh(core_axis_name="c", subcore_axis_name="s",
                                  num_cores=NC, num_subcores=NS)
    return pl.kernel(
        _pool_body,
        out_type=jax.ShapeDtypeStruct((N, EMB), jnp.float32),
        mesh=mesh,
        compiler_params=pltpu.CompilerParams(needs_layout_passes=False),
        scratch_types=[
            pltpu.VMEM((TPW, N), jnp.int32),
            pltpu.VMEM((TPW, MW), jnp.int32),
            pltpu.VMEM((IW,), jnp.int32),
            pltpu.VMEM((IW,), jnp.int32),
            pltpu.VMEM((NB, CH, PW), jnp.float32),
            pltpu.VMEM((TPW, EMB), jnp.float32),
            pltpu.SemaphoreType.DMA,
        ],
    )(ridx, meta, p)


def _lstm_body(emb_ref, pool_ref, h_ref, c_ref, wih_ref, whh_ref,
               bias_ref, bsoc_ref, wout_ref, bout_ref, out_ref):
    hp = jnp.maximum(pool_ref[...] + bsoc_ref[...], 0.0)
    lstm_in = jnp.concatenate([emb_ref[...], hp], axis=1)
    gates = (jnp.dot(lstm_in, wih_ref[...], preferred_element_type=jnp.float32)
             + jnp.dot(h_ref[...], whh_ref[...],
                       preferred_element_type=jnp.float32)
             + bias_ref[...])
    i_g = gates[:, 0:RNN]
    f_g = gates[:, RNN:2 * RNN]
    g_g = gates[:, 2 * RNN:3 * RNN]
    o_g = gates[:, 3 * RNN:4 * RNN]
    c_new = (jax.nn.sigmoid(f_g) * c_ref[...]
             + jax.nn.sigmoid(i_g) * jnp.tanh(g_g))
    h_new = jax.nn.sigmoid(o_g) * jnp.tanh(c_new)
    out_ref[...] = (jnp.dot(h_new, wout_ref[...],
                            preferred_element_type=jnp.float32)
                    + bout_ref[...])


def _lstm(emb, pool, h, c, wih_t, whh_t, bias, bsoc, wout_p, bout_p):
    return pl.pallas_call(
        _lstm_body,
        out_shape=jax.ShapeDtypeStruct((N, 128), jnp.float32),
    )(emb, pool, h, c, wih_t, whh_t, bias, bsoc, wout_p, bout_p)


def kernel(xoff, xabs, h0, c0, W_embed, b_embed, W_social, b_social,
           W_ih, W_hh, b_ih, b_hh, W_out, b_out):
    h = h0[0]
    c = c0[0]
    wr = jnp.pad(W_social.reshape(GG, RNN, EMB), ((0, 0), (0, 0), (0, PW - EMB)))
    xsc = xabs[:, 0:1]
    ysc = xabs[:, 1:2]
    xsr = xabs[:, 0].reshape(1, N)
    ysr = xabs[:, 1].reshape(1, N)

    p, emb, ridx, meta = _prep(h, wr, xoff, W_embed, b_embed.reshape(1, EMB),
                               xsc, ysc, xsr, ysr)

    pool = _pool(ridx, meta, p)

    bias = (b_ih + b_hh).reshape(1, 4 * RNN)
    wout_p = jnp.pad(W_out, ((0, 0), (0, 128 - OUTD)))
    bout_p = jnp.pad(b_out, (0, 128 - OUTD)).reshape(1, 128)
    final = _lstm(emb, pool, h, c, W_ih.T, W_hh.T, bias,
                  b_social.reshape(1, EMB), wout_p, bout_p)[:, :OUTD]

    mu1, mu2, log_s1, log_s2, rho, pi = jnp.split(final, 6, axis=1)
    return (mu1, mu2, log_s1, log_s2, rho, pi)


# speedup vs baseline: 7.5068x; 1.1619x over previous
import functools

import numpy as np
import jax
import jax.numpy as jnp
from jax import lax
from jax.experimental import pallas as pl
from jax.experimental.pallas import tpu as pltpu
from jax.experimental.pallas import tpu_sc as plsc

N = 512
EMB = 64
RNN = 128
GRID = 8
GG = GRID * GRID
NMIX = 20
OUTD = NMIX * 6
NEIGH = 0.4

NC, NS, L = 2, 16, 16
NW = NC * NS
TPW = N // NW
PROWS = GG * N
PPAD = PROWS + N
CH = 16
CHSH = 4
NBH = 4
MW = N + 2 * L + 96
TRASH = N + CH
IW = N + CH + L
PW = 128


def _prep_body(h_ref, wr_ref, xoff_ref, wemb_ref, bemb_ref,
               xsc_ref, ysc_ref, xsr_ref, ysr_ref,
               p_ref, emb_ref, ridx_ref, meta_ref):
    c = pl.program_id(0)

    @pl.when(c < GG)
    def _():
        p_ref[...] = jnp.dot(h_ref[...], wr_ref[0],
                             preferred_element_type=jnp.float32)

    @pl.when(c == GG)
    def _():
        p_ref[...] = jnp.zeros_like(p_ref)

    @pl.when(c == 0)
    def _():
        xo = xoff_ref[...]
        w = wemb_ref[...]
        emb = xo[:, 0:1] * w[0:1, :] + xo[:, 1:2] * w[1:2, :] + bemb_ref[...]
        emb_ref[...] = jnp.maximum(emb, 0.0)

        dx = xsr_ref[...] - (xsc_ref[...] - NEIGH / 2.0)
        dy = ysr_ref[...] - (ysc_ref[...] - NEIGH / 2.0)
        within = (dx >= 0.0) & (dx < NEIGH) & (dy >= 0.0) & (dy < NEIGH)
        cellx = jnp.floor(dx / NEIGH * GRID).astype(jnp.int32)
        celly = jnp.floor(dy / NEIGH * GRID).astype(jnp.int32)
        valid_cell = ((cellx >= 0) & (cellx < GRID)
                      & (celly >= 0) & (celly < GRID))
        idxc = jnp.clip(cellx + celly * GRID, 0, GG - 1)
        col = lax.broadcasted_iota(jnp.int32, (N, N), 1)
        row = lax.broadcasted_iota(jnp.int32, (N, N), 0)
        valid = within & valid_cell & (col != row)
        ridx_ref[...] = jnp.where(valid, idxc * N + col, PROWS)

        vf = valid.astype(jnp.float32)
        tri = (lax.broadcasted_iota(jnp.int32, (N, MW), 0)
               < lax.broadcasted_iota(jnp.int32, (N, MW), 1)
               ).astype(jnp.float32)
        pos = jnp.dot(vf, tri, preferred_element_type=jnp.float32)
        posi = pos.astype(jnp.int32)
        meta_ref[...] = jnp.concatenate(
            [jnp.where(valid, posi[:, :N], TRASH), posi[:, N:]], axis=1)


def _prep(h, wr, xoff, wemb, bemb, xsc, ysc, xsr, ysr):
    full = lambda s: pl.BlockSpec(s, lambda c: (0,) * len(s))
    return pl.pallas_call(
        _prep_body,
        grid=(GG + 1,),
        in_specs=[
            full((N, RNN)),
            pl.BlockSpec((1, RNN, PW), lambda c: (jnp.minimum(c, GG - 1), 0, 0)),
            full((N, 2)),
            full((2, EMB)),
            full((1, EMB)),
            full((N, 1)), full((N, 1)), full((1, N)), full((1, N)),
        ],
        out_specs=[
            pl.BlockSpec((N, PW), lambda c: (c, 0)),
            pl.BlockSpec((N, EMB), lambda c: (0, 0)),
            pl.BlockSpec((N, N), lambda c: (0, 0)),
            pl.BlockSpec((N, MW), lambda c: (0, 0)),
        ],
        out_shape=[
            jax.ShapeDtypeStruct((PPAD, PW), jnp.float32),
            jax.ShapeDtypeStruct((N, EMB), jnp.float32),
            jax.ShapeDtypeStruct((N, N), jnp.int32),
            jax.ShapeDtypeStruct((N, MW), jnp.int32),
        ],
    )(h, wr, xoff, wemb, bemb, xsc, ysc, xsr, ysr)


def _pool_body(ridx_hbm, meta_hbm, p_hbm, out_hbm,
               rid_v, pos_v, idx_v, rows_v, acc_v, sem0, sem1):
    wid = lax.axis_index("s") * NC + lax.axis_index("c")
    base = wid * TPW
    pltpu.sync_copy(ridx_hbm.at[pl.ds(base, TPW)], rid_v)
    pltpu.sync_copy(meta_hbm.at[pl.ds(base, TPW)], pos_v)

    sems = [sem0, sem1]
    nits = [None] * TPW

    def compact(t):
        def cbody(ch, _):
            r = rid_v[t, pl.ds(ch * L, L)]
            pv = pos_v[t, pl.ds(ch * L, L)] + (t * IW)
            plsc.store_scatter(idx_v, [pv], r)
            return 0

        lax.fori_loop(0, N // L, cbody, 0, unroll=False)
        cnt = pos_v[t, pl.ds(N, L)][0]
        idx_v[pl.ds(t * IW + cnt, L)] = (jnp.full((L,), PROWS, jnp.int32)
                                         + (base + t))
        nits[t] = jnp.right_shift(cnt + CH - 1, CHSH)

    def fire(t, g, b):
        pltpu.async_copy(p_hbm.at[idx_v.at[pl.ds(t * IW + g * CH, CH)]],
                         rows_v.at[b], sems[t & 1])

    def fire_first(t):
        for b in range(NBH):
            @pl.when(b < nits[t])
            def _(t=t, b=b):
                fire(t, b, (t & 1) * NBH + b)

    def drain(t):
        nit = nits[t]

        def gbody(g, accs):
            pltpu.make_async_copy(
                p_hbm.at[idx_v.at[pl.ds(0, CH)]],
                rows_v.at[0], sems[t & 1]).wait()
            b = (t & 1) * NBH + jnp.bitwise_and(g, NBH - 1)

            @pl.when(g + NBH < nit)
            def _():
                fire(t, g + NBH, b)

            def abody(r, accs2):
                b0, b1, b2, b3 = accs2
                return (b0 + rows_v[b, r, pl.ds(0, L)],
                        b1 + rows_v[b, r, pl.ds(L, L)],
                        b2 + rows_v[b, r, pl.ds(2 * L, L)],
                        b3 + rows_v[b, r, pl.ds(3 * L, L)])

            return lax.fori_loop(0, CH, abody, accs, unroll=False)

        zero = jnp.zeros((L,), jnp.float32)
        a0, a1, a2, a3 = lax.fori_loop(0, nit, gbody,
                                       (zero, zero, zero, zero),
                                       unroll=False)
        acc_v[t, pl.ds(0, L)] = a0
        acc_v[t, pl.ds(L, L)] = a1
        acc_v[t, pl.ds(2 * L, L)] = a2
        acc_v[t, pl.ds(3 * L, L)] = a3

    compact(0)
    fire_first(0)
    compact(1)
    fire_first(1)
    for t in range(TPW):
        if t + 2 < TPW:
            compact(t + 2)
        drain(t)
        if t + 2 < TPW:
            fire_first(t + 2)

    pltpu.sync_copy(acc_v, out_hbm.at[pl.ds(base, TPW)])


def _pool(ridx, meta, p):
    mesh = plsc.VectorSubcoreMesh(core_axis_name="c", subcore_axis_name="s",
                                  num_cores=NC, num_subcores=NS)
    return pl.kernel(
        _pool_body,
        out_type=jax.ShapeDtypeStruct((N, EMB), jnp.float32),
        mesh=mesh,
        compiler_params=pltpu.CompilerParams(needs_layout_passes=False),
        scratch_types=[
            pltpu.VMEM((TPW, N), jnp.int32),
            pltpu.VMEM((TPW, MW), jnp.int32),
            pltpu.VMEM((TPW * IW, ), jnp.int32),
            pltpu.VMEM((2 * NBH, CH, PW), jnp.float32),
            pltpu.VMEM((TPW, EMB), jnp.float32),
            pltpu.SemaphoreType.DMA,
            pltpu.SemaphoreType.DMA,
        ],
    )(ridx, meta, p)


def _lstm_body(emb_ref, pool_ref, h_ref, c_ref, wih_ref, whh_ref,
               bias_ref, bsoc_ref, wout_ref, bout_ref, out_ref):
    hp = jnp.maximum(pool_ref[...] + bsoc_ref[...], 0.0)
    lstm_in = jnp.concatenate([emb_ref[...], hp], axis=1)
    gates = (jnp.dot(lstm_in, wih_ref[...], preferred_element_type=jnp.float32)
             + jnp.dot(h_ref[...], whh_ref[...],
                       preferred_element_type=jnp.float32)
             + bias_ref[...])
    i_g = gates[:, 0:RNN]
    f_g = gates[:, RNN:2 * RNN]
    g_g = gates[:, 2 * RNN:3 * RNN]
    o_g = gates[:, 3 * RNN:4 * RNN]
    c_new = (jax.nn.sigmoid(f_g) * c_ref[...]
             + jax.nn.sigmoid(i_g) * jnp.tanh(g_g))
    h_new = jax.nn.sigmoid(o_g) * jnp.tanh(c_new)
    out_ref[...] = (jnp.dot(h_new, wout_ref[...],
                            preferred_element_type=jnp.float32)
                    + bout_ref[...])


def _lstm(emb, pool, h, c, wih_t, whh_t, bias, bsoc, wout_p, bout_p):
    return pl.pallas_call(
        _lstm_body,
        out_shape=jax.ShapeDtypeStruct((N, 128), jnp.float32),
    )(emb, pool, h, c, wih_t, whh_t, bias, bsoc, wout_p, bout_p)


def kernel(xoff, xabs, h0, c0, W_embed, b_embed, W_social, b_social,
           W_ih, W_hh, b_ih, b_hh, W_out, b_out):
    h = h0[0]
    c = c0[0]
    wr = jnp.pad(W_social.reshape(GG, RNN, EMB), ((0, 0), (0, 0), (0, PW - EMB)))
    xsc = xabs[:, 0:1]
    ysc = xabs[:, 1:2]
    xsr = xabs[:, 0].reshape(1, N)
    ysr = xabs[:, 1].reshape(1, N)

    p, emb, ridx, meta = _prep(h, wr, xoff, W_embed, b_embed.reshape(1, EMB),
                               xsc, ysc, xsr, ysr)

    pool = _pool(ridx, meta, p)

    bias = (b_ih + b_hh).reshape(1, 4 * RNN)
    wout_p = jnp.pad(W_out, ((0, 0), (0, 128 - OUTD)))
    bout_p = jnp.pad(b_out, (0, 128 - OUTD)).reshape(1, 128)
    final = _lstm(emb, pool, h, c, W_ih.T, W_hh.T, bias,
                  b_social.reshape(1, EMB), wout_p, bout_p)[:, :OUTD]

    mu1, mu2, log_s1, log_s2, rho, pi = jnp.split(final, 6, axis=1)
    return (mu1, mu2, log_s1, log_s2, rho, pi)
